# Initial kernel scaffold; baseline (speedup 1.0000x reference)
import jax
import jax.numpy as jnp
from jax.experimental import pallas as pl


def kernel(node_feature, edge_state, edge_dst, W, b):
    raise NotImplementedError("write your pallas kernel here")


# speedup vs baseline: 3.7474x; 3.7474x over previous
import functools

import jax
import jax.numpy as jnp
from jax import lax
from jax.experimental import pallas as pl
from jax.experimental.pallas import tpu as pltpu
from jax.experimental.pallas import tpu_sc as plsc

NC = 2
NS = 16
NW = NC * NS

CHUNK = 80
ZROWS = 32


def _segment_sum_sc(edge_state, edge_dst, n_pad):
    n_edges, d = edge_state.shape
    assert n_edges % NW == 0
    epw = n_edges // NW
    assert epw % CHUNK == 0
    nchunk = epw // CHUNK
    rows_per_tile = n_pad // NS
    assert rows_per_tile * NS == n_pad and rows_per_tile % 8 == 0
    assert rows_per_tile % ZROWS == 0
    nzcopy = rows_per_tile // ZROWS

    mesh = plsc.VectorSubcoreMesh(core_axis_name="c", subcore_axis_name="s")

    @functools.partial(
        pl.kernel,
        out_type=jax.ShapeDtypeStruct((NC, n_pad, d), jnp.float32),
        mesh=mesh,
        scratch_types=[
            pltpu.VMEM_SHARED((n_pad, d), jnp.float32),
            pltpu.VMEM((CHUNK, d), jnp.float32),
            pltpu.VMEM((CHUNK,), jnp.int32),
            pltpu.VMEM((ZROWS, d), jnp.float32),
        ],
    )
    def seg_sum(es_hbm, dst_hbm, out_hbm, acc, rows_v, idx_v, zbuf):
        cid = lax.axis_index("c")
        sid = lax.axis_index("s")
        wid = sid * NC + cid

        zero16 = jnp.zeros((16,), jnp.float32)
        lanes = d // 16

        def zstore(i, carry):
            zbuf[i // lanes, pl.ds((i % lanes) * 16, 16)] = zero16
            return carry

        lax.fori_loop(0, ZROWS * lanes, zstore, 0)

        def zcopy(k, carry):
            pltpu.sync_copy(zbuf, acc.at[pl.ds(sid * rows_per_tile + k * ZROWS, ZROWS)])
            return carry

        lax.fori_loop(0, nzcopy, zcopy, 0)
        plsc.subcore_barrier()

        def body(c, carry):
            base = wid * epw + c * CHUNK
            pltpu.sync_copy(es_hbm.at[pl.ds(base, CHUNK)], rows_v)
            pltpu.sync_copy(dst_hbm.at[pl.ds(base, CHUNK)], idx_v)
            pltpu.sync_copy(rows_v, acc.at[idx_v], add=True)
            return carry

        lax.fori_loop(0, nchunk, body, 0)
        plsc.subcore_barrier()

        pltpu.sync_copy(
            acc.at[pl.ds(sid * rows_per_tile, rows_per_tile)],
            out_hbm.at[cid, pl.ds(sid * rows_per_tile, rows_per_tile)],
        )

    return seg_sum(edge_state, edge_dst)


def _dense_body(nf_ref, p_ref, w_ref, b_ref, o_ref):
    d = nf_ref.shape[1]
    ns = p_ref[0] + p_ref[1]
    x = jnp.dot(nf_ref[...], w_ref[0:d, :], preferred_element_type=jnp.float32)
    y = jnp.dot(ns, w_ref[d:, :], preferred_element_type=jnp.float32)
    o_ref[...] = jnp.maximum(x + y + b_ref[...], 0.0)


def kernel(node_feature, edge_state, edge_dst, W, b):
    n_nodes, d = node_feature.shape
    units = W.shape[1]
    g = NS * ZROWS
    n_pad = ((n_nodes + g - 1) // g) * g
    partials = _segment_sum_sc(edge_state, edge_dst.astype(jnp.int32), n_pad)

    blk = 1000
    assert n_nodes % blk == 0
    grid = (n_nodes // blk,)
    out = pl.pallas_call(
        _dense_body,
        grid=grid,
        in_specs=[
            pl.BlockSpec((blk, d), lambda i: (i, 0)),
            pl.BlockSpec((NC, blk, d), lambda i: (0, i, 0)),
            pl.BlockSpec(W.shape, lambda i: (0, 0)),
            pl.BlockSpec((1, units), lambda i: (0, 0)),
        ],
        out_specs=pl.BlockSpec((blk, units), lambda i: (i, 0)),
        out_shape=jax.ShapeDtypeStruct((n_nodes, units), jnp.float32),
    )(node_feature, partials, W, b.reshape(1, units))
    return out


# speedup vs baseline: 5.8853x; 1.5705x over previous
import functools

import jax
import jax.numpy as jnp
from jax import lax
from jax.experimental import pallas as pl
from jax.experimental.pallas import tpu as pltpu
from jax.experimental.pallas import tpu_sc as plsc

NC = 2
NS = 16
NW = NC * NS

CHUNK = 80
ZROWS = 32


def _segment_sum_sc(edge_state, edge_dst, n_pad):
    n_edges, d = edge_state.shape
    assert n_edges % NW == 0
    epw = n_edges // NW
    assert epw % CHUNK == 0
    nchunk = epw // CHUNK
    lchunk = CHUNK
    spl = lchunk // CHUNK
    assert epw % lchunk == 0
    nload = epw // lchunk
    assert nload % 2 == 1 and nload >= 3
    npair = nload // 2
    rows_per_tile = n_pad // NS
    assert rows_per_tile * NS == n_pad and rows_per_tile % 8 == 0
    assert rows_per_tile % ZROWS == 0
    nzcopy = rows_per_tile // ZROWS

    mesh = plsc.VectorSubcoreMesh(core_axis_name="c", subcore_axis_name="s")

    @functools.partial(
        pl.kernel,
        out_type=jax.ShapeDtypeStruct((NC, n_pad, d), jnp.float32),
        mesh=mesh,
        scratch_types=[
            pltpu.VMEM_SHARED((n_pad, d), jnp.float32),
            pltpu.VMEM((lchunk, d), jnp.float32),
            pltpu.VMEM((lchunk, d), jnp.float32),
            pltpu.VMEM((nchunk, CHUNK), jnp.int32),
            pltpu.VMEM((ZROWS, d), jnp.float32),
            pltpu.SemaphoreType.DMA,
            pltpu.SemaphoreType.DMA,
        ],
    )
    def seg_sum(es_hbm, dst3_hbm, out_hbm, acc, rows_a, rows_b, idx2, zbuf,
                sem_a, sem_b):
        cid = lax.axis_index("c")
        sid = lax.axis_index("s")
        wid = sid * NC + cid

        idx_cp = pltpu.async_copy(dst3_hbm.at[wid], idx2, sem_a)

        zero16 = jnp.zeros((16,), jnp.float32)
        lanes = d // 16

        def zstore(i, carry):
            zbuf[i // lanes, pl.ds((i % lanes) * 16, 16)] = zero16
            return carry

        lax.fori_loop(0, ZROWS * lanes, zstore, 0)

        def zcopy(k, carry):
            pltpu.sync_copy(zbuf, acc.at[pl.ds(sid * rows_per_tile + k * ZROWS, ZROWS)])
            return carry

        lax.fori_loop(0, nzcopy, zcopy, 0)
        idx_cp.wait()
        plsc.subcore_barrier()

        def start_load(li, buf, sem):
            base = wid * epw + li * lchunk
            pltpu.async_copy(es_hbm.at[pl.ds(base, lchunk)], buf, sem)

        def wait_load(buf, sem):
            pltpu.make_async_copy(es_hbm.at[pl.ds(0, lchunk)], buf, sem).wait()

        def scatter_block(buf, li):
            for s in range(spl):
                pltpu.sync_copy(
                    buf.at[pl.ds(s * CHUNK, CHUNK)],
                    acc.at[idx2.at[li * spl + s]],
                    add=True,
                )

        start_load(0, rows_a, sem_a)

        def body(j, carry):
            wait_load(rows_a, sem_a)
            start_load(2 * j + 1, rows_b, sem_b)
            scatter_block(rows_a, 2 * j)
            wait_load(rows_b, sem_b)
            start_load(2 * j + 2, rows_a, sem_a)
            scatter_block(rows_b, 2 * j + 1)
            return carry

        lax.fori_loop(0, npair, body, 0)
        wait_load(rows_a, sem_a)
        scatter_block(rows_a, nload - 1)
        plsc.subcore_barrier()

        pltpu.sync_copy(
            acc.at[pl.ds(sid * rows_per_tile, rows_per_tile)],
            out_hbm.at[cid, pl.ds(sid * rows_per_tile, rows_per_tile)],
        )

    return seg_sum(edge_state, edge_dst.reshape(NW, nchunk, CHUNK))


def _dense_body(nf_ref, p_ref, w_ref, b_ref, o_ref):
    d = nf_ref.shape[1]
    ns = p_ref[0] + p_ref[1]
    x = jnp.dot(nf_ref[...], w_ref[0:d, :], preferred_element_type=jnp.float32)
    y = jnp.dot(ns, w_ref[d:, :], preferred_element_type=jnp.float32)
    o_ref[...] = jnp.maximum(x + y + b_ref[...], 0.0)


def kernel(node_feature, edge_state, edge_dst, W, b):
    n_nodes, d = node_feature.shape
    units = W.shape[1]
    g = NS * ZROWS
    n_pad = ((n_nodes + g - 1) // g) * g
    partials = _segment_sum_sc(edge_state, edge_dst.astype(jnp.int32), n_pad)

    blk = 1000
    assert n_nodes % blk == 0
    grid = (n_nodes // blk,)
    out = pl.pallas_call(
        _dense_body,
        grid=grid,
        in_specs=[
            pl.BlockSpec((blk, d), lambda i: (i, 0)),
            pl.BlockSpec((NC, blk, d), lambda i: (0, i, 0)),
            pl.BlockSpec(W.shape, lambda i: (0, 0)),
            pl.BlockSpec((1, units), lambda i: (0, 0)),
        ],
        out_specs=pl.BlockSpec((blk, units), lambda i: (i, 0)),
        out_shape=jax.ShapeDtypeStruct((n_nodes, units), jnp.float32),
    )(node_feature, partials, W, b.reshape(1, units))
    return out
